# baseline (speedup 1.0000x reference)
import functools

import jax
import jax.numpy as jnp
from jax import lax
from jax.experimental import pallas as pl
from jax.experimental.pallas import tpu as pltpu
from jax.experimental.pallas import tpu_sc as plsc

N = 10000
D = 128
H = 128
T = 4
E = 320000
P = 100000

NC = 2
NS = 16
LANES = 16

NP = 10240
EP = 327680
PP = 102400

EC = 128
E_TILE = EP // NS
E_CHUNKS = E_TILE // EC
ROWS_TILE = NP // NS
P_TILE = PP // (NC * NS)
P_CHUNKS = P_TILE // EC

SC_ = 40
S_CHUNKS = E_TILE // SC_
S_GRP = 32
S_RING = 8

BN = 1024


_DEG_GRP = 8
_GRP = 32


def _deg_body(dst_hbm, ones_hbm, deg_hbm, didx_all, ones_v, acc_sh, sem):
  c = lax.axis_index("c")
  s = lax.axis_index("s")
  pltpu.sync_copy(ones_hbm.at[pl.ds(0, EC)], ones_v)

  for tt in range(T // NC):
    t = c * (T // NC) + tt
    pltpu.sync_copy(dst_hbm.at[t, s], didx_all)
    pltpu.sync_copy(ones_hbm.at[pl.ds(s * ROWS_TILE, ROWS_TILE)],
                    acc_sh.at[pl.ds(s * ROWS_TILE, ROWS_TILE)])
    plsc.subcore_barrier()

    def group(g, carry):
      for k in range(_DEG_GRP):
        pltpu.async_copy(ones_v, acc_sh.at[didx_all.at[g * _DEG_GRP + k]],
                         sem, add=True)
      for k in range(_DEG_GRP):
        pltpu.make_async_copy(ones_v, acc_sh.at[didx_all.at[0]],
                              sem).wait()
      return carry

    lax.fori_loop(0, E_CHUNKS // _DEG_GRP, group, 0)
    plsc.subcore_barrier()
    pltpu.sync_copy(acc_sh.at[pl.ds(s * ROWS_TILE, ROWS_TILE)],
                    deg_hbm.at[t, pl.ds(s * ROWS_TILE, ROWS_TILE)])
    plsc.subcore_barrier()


def _spmm_body(u_hbm, src_hbm, dst_hbm, agg_hbm, sidx_g, didx_g,
               r0, r1, r2, r3, r4, r5, r6, r7, acc_sh,
               g0, g1, g2, g3, g4, g5, g6, g7,
               s0, s1, s2, s3, s4, s5, s6, s7):
  c = lax.axis_index("c")
  s = lax.axis_index("s")
  rows = [r0, r1, r2, r3, r4, r5, r6, r7]
  gsem = [g0, g1, g2, g3, g4, g5, g6, g7]
  ssem = [s0, s1, s2, s3, s4, s5, s6, s7]

  for tt in range(T // NC):
    t = c * (T // NC) + tt
    pltpu.sync_copy(u_hbm.at[t, pl.ds(s * ROWS_TILE, ROWS_TILE)],
                    acc_sh.at[pl.ds(s * ROWS_TILE, ROWS_TILE)])
    plsc.subcore_barrier()

    ut = u_hbm.at[t]

    def fire_gather(lj, k):
      pltpu.async_copy(ut.at[sidx_g.at[lj]], rows[k], gsem[k])

    def wait_gather(k):
      pltpu.make_async_copy(ut.at[sidx_g.at[0]], rows[k], gsem[k]).wait()

    def fire_scatter(lj, k):
      pltpu.async_copy(rows[k], acc_sh.at[didx_g.at[lj]], ssem[k], add=True)

    def wait_scatter(k):
      pltpu.make_async_copy(rows[k], acc_sh.at[didx_g.at[0]],
                            ssem[k]).wait()

    def group(gg, carry):
      pltpu.sync_copy(src_hbm.at[t, s, pl.ds(gg * S_GRP, S_GRP)], sidx_g)
      pltpu.sync_copy(dst_hbm.at[t, s, pl.ds(gg * S_GRP, S_GRP)], didx_g)
      for k in range(S_RING):
        fire_gather(k, k)

      def inner(ii, carry2):
        base = S_RING * ii
        for k in range(S_RING):
          wait_gather(k)
          fire_scatter(base + k, k)
        for k in range(S_RING):
          wait_scatter(k)
          fire_gather(base + S_RING + k, k)
        return carry2

      lax.fori_loop(0, S_GRP // S_RING - 1, inner, 0)
      for k in range(S_RING):
        wait_gather(k)
        fire_scatter(S_GRP - S_RING + k, k)
      for k in range(S_RING):
        wait_scatter(k)
      return carry

    lax.fori_loop(0, S_CHUNKS // S_GRP, group, 0)
    plsc.subcore_barrier()
    pltpu.sync_copy(acc_sh.at[pl.ds(s * ROWS_TILE, ROWS_TILE)],
                    agg_hbm.at[t, pl.ds(s * ROWS_TILE, ROWS_TILE)])
    plsc.subcore_barrier()


def _score_body(pred_hbm, ps_hbm, pd_hbm, out_hbm, aidx, bidx, arows, brows,
                outv16, sem):
  c = lax.axis_index("c")
  s = lax.axis_index("s")
  w = s * NC + c

  def chunk(j, carry):
    base = w * P_TILE + j * EC
    pltpu.sync_copy(ps_hbm.at[pl.ds(base, EC)], aidx)
    pltpu.sync_copy(pd_hbm.at[pl.ds(base, EC)], bidx)
    pltpu.async_copy(pred_hbm.at[aidx], arows, sem).wait()
    pltpu.async_copy(pred_hbm.at[bidx], brows, sem).wait()

    def pair(p, carry2):
      acc = arows[p, pl.ds(0, LANES)] * brows[p, pl.ds(0, LANES)]
      for q in range(1, H // LANES):
        acc = acc + arows[p, pl.ds(q * LANES, LANES)] * \
            brows[p, pl.ds(q * LANES, LANES)]
      outv16[p, :] = acc
      return carry2

    lax.fori_loop(0, EC, pair, 0)
    pltpu.sync_copy(outv16, out_hbm.at[pl.ds(base, EC)])
    return carry

  lax.fori_loop(0, P_CHUNKS, chunk, 0)


_sc_mesh = plsc.VectorSubcoreMesh(core_axis_name="c", subcore_axis_name="s")

_deg_kernel = pl.kernel(
    _deg_body,
    out_type=jax.ShapeDtypeStruct((T, NP, H), jnp.float32),
    mesh=_sc_mesh,
    scratch_types=[
        pltpu.VMEM((E_CHUNKS, EC), jnp.int32),
        pltpu.VMEM((EC, H), jnp.float32),
        pltpu.VMEM_SHARED((NP, H), jnp.float32),
        pltpu.SemaphoreType.DMA,
    ],
)

_spmm_kernel = pl.kernel(
    _spmm_body,
    out_type=jax.ShapeDtypeStruct((T, NP, H), jnp.float32),
    mesh=_sc_mesh,
    scratch_types=[
        pltpu.VMEM((S_GRP, SC_), jnp.int32),
        pltpu.VMEM((S_GRP, SC_), jnp.int32),
    ] + [pltpu.VMEM((SC_, H), jnp.float32)] * 8
    + [pltpu.VMEM_SHARED((NP, H), jnp.float32)]
    + [pltpu.SemaphoreType.DMA] * 16,
)

_score_kernel = pl.kernel(
    _score_body,
    out_type=jax.ShapeDtypeStruct((PP, 16), jnp.float32),
    mesh=_sc_mesh,
    scratch_types=[
        pltpu.VMEM((EC,), jnp.int32),
        pltpu.VMEM((EC,), jnp.int32),
        pltpu.VMEM((EC, H), jnp.float32),
        pltpu.VMEM((EC, H), jnp.float32),
        pltpu.VMEM((EC, 16), jnp.float32),
        pltpu.SemaphoreType.DMA,
    ],
)


def _dis_from_deg(deg_blk):
  deg = deg_blk[:, 0:1]
  return lax.rsqrt(jnp.maximum(deg, 1e-12)).astype(jnp.float32)


def _tc_first_body(x_ref, deg_ref, w_ref, u_ref):
  dis = _dis_from_deg(deg_ref[...])
  h = jnp.dot(x_ref[...], w_ref[...], preferred_element_type=jnp.float32)
  u_ref[...] = dis * h


def _tc_layer_body(agg_ref, deg_ref, w_ref, b_ref, u_ref):
  dis = _dis_from_deg(deg_ref[...])
  h = jax.nn.relu(dis * agg_ref[...] + b_ref[...])
  u_ref[...] = dis * jnp.dot(h, w_ref[...], preferred_element_type=jnp.float32)


def _tc_gru_body(agg_ref, deg_ref, b3_ref, wih_ref, whh_ref, bih_ref, bhh_ref,
                 pred_ref):
  h = jnp.zeros((BN, H), jnp.float32)
  for t in range(T):
    dis = _dis_from_deg(deg_ref[t])
    xt = dis * agg_ref[t] + b3_ref[...]
    gi = jnp.dot(xt, wih_ref[...], preferred_element_type=jnp.float32) \
        + bih_ref[...]
    gh = jnp.dot(h, whh_ref[...], preferred_element_type=jnp.float32) \
        + bhh_ref[...]
    i_r, i_z, i_n = gi[:, 0:H], gi[:, H:2 * H], gi[:, 2 * H:3 * H]
    h_r, h_z, h_n = gh[:, 0:H], gh[:, H:2 * H], gh[:, 2 * H:3 * H]
    r = jax.nn.sigmoid(i_r + h_r)
    z = jax.nn.sigmoid(i_z + h_z)
    n = jnp.tanh(i_n + r * h_n)
    h = (1.0 - z) * n + z * h
  pred_ref[...] = h


_ROWS = T * NP // BN

_tc_first = pl.pallas_call(
    _tc_first_body,
    grid=(_ROWS,),
    in_specs=[
        pl.BlockSpec((BN, D), lambda i: (i, 0)),
        pl.BlockSpec((BN, H), lambda i: (i, 0)),
        pl.BlockSpec((D, H), lambda i: (0, 0)),
    ],
    out_specs=pl.BlockSpec((BN, H), lambda i: (i, 0)),
    out_shape=jax.ShapeDtypeStruct((T * NP, H), jnp.float32),
    compiler_params=pltpu.CompilerParams(
        dimension_semantics=("parallel",)),
)

_tc_layer = pl.pallas_call(
    _tc_layer_body,
    grid=(_ROWS,),
    in_specs=[
        pl.BlockSpec((BN, H), lambda i: (i, 0)),
        pl.BlockSpec((BN, H), lambda i: (i, 0)),
        pl.BlockSpec((H, H), lambda i: (0, 0)),
        pl.BlockSpec((1, H), lambda i: (0, 0)),
    ],
    out_specs=pl.BlockSpec((BN, H), lambda i: (i, 0)),
    out_shape=jax.ShapeDtypeStruct((T * NP, H), jnp.float32),
    compiler_params=pltpu.CompilerParams(
        dimension_semantics=("parallel",)),
)

_BR = 80


def _tc_red_body(s16_ref, out_ref):
  out_ref[...] = jnp.sum(s16_ref[...], axis=-1)


_tc_red = pl.pallas_call(
    _tc_red_body,
    grid=(PP // 128 // _BR,),
    in_specs=[pl.BlockSpec((_BR, 128, 16), lambda i: (i, 0, 0))],
    out_specs=pl.BlockSpec((_BR, 128), lambda i: (i, 0)),
    out_shape=jax.ShapeDtypeStruct((PP // 128, 128), jnp.float32),
    compiler_params=pltpu.CompilerParams(
        dimension_semantics=("parallel",)),
)

_tc_gru = pl.pallas_call(
    _tc_gru_body,
    grid=(NP // BN,),
    in_specs=[
        pl.BlockSpec((T, BN, H), lambda i: (0, i, 0)),
        pl.BlockSpec((T, BN, H), lambda i: (0, i, 0)),
        pl.BlockSpec((1, H), lambda i: (0, 0)),
        pl.BlockSpec((H, 3 * H), lambda i: (0, 0)),
        pl.BlockSpec((H, 3 * H), lambda i: (0, 0)),
        pl.BlockSpec((1, 3 * H), lambda i: (0, 0)),
        pl.BlockSpec((1, 3 * H), lambda i: (0, 0)),
    ],
    out_specs=pl.BlockSpec((BN, H), lambda i: (i, 0)),
    out_shape=jax.ShapeDtypeStruct((NP, H), jnp.float32),
    compiler_params=pltpu.CompilerParams(
        dimension_semantics=("parallel",)),
)


@jax.jit
def _run(x_seq, edge_index, edge_pairs, W1, b1, W2, b2, W3, b3, W_ih, W_hh,
         b_ih, b_hh):
  x = jnp.pad(x_seq.astype(jnp.float32), ((0, 0), (0, NP - N), (0, 0)))
  ei = edge_index.astype(jnp.int32)
  src = jnp.pad(ei[:, 0, :], ((0, 0), (0, EP - E)), constant_values=N)
  dst = jnp.pad(ei[:, 1, :], ((0, 0), (0, EP - E)), constant_values=N)
  pairs = edge_pairs.astype(jnp.int32)
  psrc = jnp.pad(pairs[0], (0, PP - P))
  pdst = jnp.pad(pairs[1], (0, PP - P))
  ones = jnp.ones((NP, H), jnp.float32)
  srcS = src.reshape(T, NS, S_CHUNKS, SC_)
  dstS = dst.reshape(T, NS, S_CHUNKS, SC_)
  dstD = dst.reshape(T, NS, E_CHUNKS, EC)

  deg = _deg_kernel(dstD, ones)
  deg_flat = deg.reshape(T * NP, H)

  u = _tc_first(x.reshape(T * NP, D), deg_flat, W1)
  agg = _spmm_kernel(u.reshape(T, NP, H), srcS, dstS)

  u = _tc_layer(agg.reshape(T * NP, H), deg_flat, W2, b1.reshape(1, H))
  agg = _spmm_kernel(u.reshape(T, NP, H), srcS, dstS)

  u = _tc_layer(agg.reshape(T * NP, H), deg_flat, W3, b2.reshape(1, H))
  agg = _spmm_kernel(u.reshape(T, NP, H), srcS, dstS)

  pred = _tc_gru(agg, deg, b3.reshape(1, H),
                 W_ih.T, W_hh.T,
                 b_ih.reshape(1, 3 * H), b_hh.reshape(1, 3 * H))

  s16 = _score_kernel(pred, psrc, pdst)
  scores = _tc_red(s16.reshape(PP // 128, 128, 16))
  return scores.reshape(PP)[:P]


def kernel(x_seq, edge_index, edge_pairs, W1, b1, W2, b2, W3, b3, W_ih, W_hh,
           b_ih, b_hh):
  return _run(x_seq, edge_index, edge_pairs, W1, b1, W2, b2, W3, b3,
              W_ih, W_hh, b_ih, b_hh)


# speedup vs baseline: 1.0129x; 1.0129x over previous
import functools

import jax
import jax.numpy as jnp
from jax import lax
from jax.experimental import pallas as pl
from jax.experimental.pallas import tpu as pltpu
from jax.experimental.pallas import tpu_sc as plsc

N = 10000
D = 128
H = 128
T = 4
E = 320000
P = 100000

NC = 2
NS = 16
LANES = 16

NP = 10240
EP = 327680
PP = 102400

EC = 128
E_TILE = EP // NS
E_CHUNKS = E_TILE // EC
ROWS_TILE = NP // NS
P_TILE = PP // (NC * NS)
P_CHUNKS = P_TILE // EC

SC_ = 80
S_CHUNKS = E_TILE // SC_
S_GRP = 32
S_RING = 4

BN = 1024


_DEG_GRP = 8
_GRP = 32


def _deg_body(dst_hbm, ones_hbm, deg_hbm, didx_all, ones_v, acc_sh, sem):
  c = lax.axis_index("c")
  s = lax.axis_index("s")
  pltpu.sync_copy(ones_hbm.at[pl.ds(0, EC)], ones_v)

  for tt in range(T // NC):
    t = c * (T // NC) + tt
    pltpu.sync_copy(dst_hbm.at[t, s], didx_all)
    pltpu.sync_copy(ones_hbm.at[pl.ds(s * ROWS_TILE, ROWS_TILE)],
                    acc_sh.at[pl.ds(s * ROWS_TILE, ROWS_TILE)])
    plsc.subcore_barrier()

    def group(g, carry):
      for k in range(_DEG_GRP):
        pltpu.async_copy(ones_v, acc_sh.at[didx_all.at[g * _DEG_GRP + k]],
                         sem, add=True)
      for k in range(_DEG_GRP):
        pltpu.make_async_copy(ones_v, acc_sh.at[didx_all.at[0]],
                              sem).wait()
      return carry

    lax.fori_loop(0, E_CHUNKS // _DEG_GRP, group, 0)
    plsc.subcore_barrier()
    pltpu.sync_copy(acc_sh.at[pl.ds(s * ROWS_TILE, ROWS_TILE)],
                    deg_hbm.at[t, pl.ds(s * ROWS_TILE, ROWS_TILE)])
    plsc.subcore_barrier()


def _spmm_body(u_hbm, src_hbm, dst_hbm, agg_hbm, sidx_g, didx_g,
               r0, r1, r2, r3, acc_sh,
               g0, g1, g2, g3, s0, s1, s2, s3):
  c = lax.axis_index("c")
  s = lax.axis_index("s")
  rows = [r0, r1, r2, r3]
  gsem = [g0, g1, g2, g3]
  ssem = [s0, s1, s2, s3]

  for tt in range(T // NC):
    t = c * (T // NC) + tt
    pltpu.sync_copy(u_hbm.at[t, pl.ds(s * ROWS_TILE, ROWS_TILE)],
                    acc_sh.at[pl.ds(s * ROWS_TILE, ROWS_TILE)])
    plsc.subcore_barrier()

    ut = u_hbm.at[t]

    def fire_gather(lj, k):
      pltpu.async_copy(ut.at[sidx_g.at[lj]], rows[k], gsem[k])

    def wait_gather(k):
      pltpu.make_async_copy(ut.at[sidx_g.at[0]], rows[k], gsem[k]).wait()

    def fire_scatter(lj, k):
      pltpu.async_copy(rows[k], acc_sh.at[didx_g.at[lj]], ssem[k], add=True)

    def wait_scatter(k):
      pltpu.make_async_copy(rows[k], acc_sh.at[didx_g.at[0]],
                            ssem[k]).wait()

    def group(gg, carry):
      pltpu.sync_copy(src_hbm.at[t, s, pl.ds(gg * S_GRP, S_GRP)], sidx_g)
      pltpu.sync_copy(dst_hbm.at[t, s, pl.ds(gg * S_GRP, S_GRP)], didx_g)
      for k in range(S_RING):
        fire_gather(k, k)

      def inner(ii, carry2):
        base = S_RING * ii
        for k in range(S_RING):
          wait_gather(k)
          fire_scatter(base + k, k)
        for k in range(S_RING):
          wait_scatter(k)
          fire_gather(base + S_RING + k, k)
        return carry2

      lax.fori_loop(0, S_GRP // S_RING - 1, inner, 0)
      for k in range(S_RING):
        wait_gather(k)
        fire_scatter(S_GRP - S_RING + k, k)
      for k in range(S_RING):
        wait_scatter(k)
      return carry

    lax.fori_loop(0, S_CHUNKS // S_GRP, group, 0)
    plsc.subcore_barrier()
    pltpu.sync_copy(acc_sh.at[pl.ds(s * ROWS_TILE, ROWS_TILE)],
                    agg_hbm.at[t, pl.ds(s * ROWS_TILE, ROWS_TILE)])
    plsc.subcore_barrier()


def _score_body(pred_hbm, ps_hbm, pd_hbm, out_hbm, aidx, bidx, arows, brows,
                outv16, sem):
  c = lax.axis_index("c")
  s = lax.axis_index("s")
  w = s * NC + c

  def chunk(j, carry):
    base = w * P_TILE + j * EC
    pltpu.sync_copy(ps_hbm.at[pl.ds(base, EC)], aidx)
    pltpu.sync_copy(pd_hbm.at[pl.ds(base, EC)], bidx)
    pltpu.async_copy(pred_hbm.at[aidx], arows, sem).wait()
    pltpu.async_copy(pred_hbm.at[bidx], brows, sem).wait()

    def pgroup(g, carry2):
      p0 = g * 8
      for k in range(8):
        p = p0 + k
        acc = arows[p, pl.ds(0, LANES)] * brows[p, pl.ds(0, LANES)]
        for q in range(1, H // LANES):
          acc = acc + arows[p, pl.ds(q * LANES, LANES)] * \
              brows[p, pl.ds(q * LANES, LANES)]
        outv16[p, :] = acc
      return carry2

    lax.fori_loop(0, EC // 8, pgroup, 0)
    pltpu.sync_copy(outv16, out_hbm.at[pl.ds(base, EC)])
    return carry

  lax.fori_loop(0, P_CHUNKS, chunk, 0)


_sc_mesh = plsc.VectorSubcoreMesh(core_axis_name="c", subcore_axis_name="s")

_deg_kernel = pl.kernel(
    _deg_body,
    out_type=jax.ShapeDtypeStruct((T, NP, H), jnp.float32),
    mesh=_sc_mesh,
    scratch_types=[
        pltpu.VMEM((E_CHUNKS, EC), jnp.int32),
        pltpu.VMEM((EC, H), jnp.float32),
        pltpu.VMEM_SHARED((NP, H), jnp.float32),
        pltpu.SemaphoreType.DMA,
    ],
)

_spmm_kernel = pl.kernel(
    _spmm_body,
    out_type=jax.ShapeDtypeStruct((T, NP, H), jnp.float32),
    mesh=_sc_mesh,
    scratch_types=[
        pltpu.VMEM((S_GRP, SC_), jnp.int32),
        pltpu.VMEM((S_GRP, SC_), jnp.int32),
    ] + [pltpu.VMEM((SC_, H), jnp.float32)] * 4
    + [pltpu.VMEM_SHARED((NP, H), jnp.float32)]
    + [pltpu.SemaphoreType.DMA] * 8,
)

_score_kernel = pl.kernel(
    _score_body,
    out_type=jax.ShapeDtypeStruct((PP, 16), jnp.float32),
    mesh=_sc_mesh,
    scratch_types=[
        pltpu.VMEM((EC,), jnp.int32),
        pltpu.VMEM((EC,), jnp.int32),
        pltpu.VMEM((EC, H), jnp.float32),
        pltpu.VMEM((EC, H), jnp.float32),
        pltpu.VMEM((EC, 16), jnp.float32),
        pltpu.SemaphoreType.DMA,
    ],
)


def _dis_from_deg(deg_blk):
  deg = deg_blk[:, 0:1]
  return lax.rsqrt(jnp.maximum(deg, 1e-12)).astype(jnp.float32)


def _tc_first_body(x_ref, deg_ref, w_ref, u_ref):
  dis = _dis_from_deg(deg_ref[...])
  h = jnp.dot(x_ref[...], w_ref[...], preferred_element_type=jnp.float32)
  u_ref[...] = dis * h


def _tc_layer_body(agg_ref, deg_ref, w_ref, b_ref, u_ref):
  dis = _dis_from_deg(deg_ref[...])
  h = jax.nn.relu(dis * agg_ref[...] + b_ref[...])
  u_ref[...] = dis * jnp.dot(h, w_ref[...], preferred_element_type=jnp.float32)


def _tc_gru_body(agg_ref, deg_ref, b3_ref, wih_ref, whh_ref, bih_ref, bhh_ref,
                 pred_ref):
  h = jnp.zeros((BN, H), jnp.float32)
  for t in range(T):
    dis = _dis_from_deg(deg_ref[t])
    xt = dis * agg_ref[t] + b3_ref[...]
    gi = jnp.dot(xt, wih_ref[...], preferred_element_type=jnp.float32) \
        + bih_ref[...]
    gh = jnp.dot(h, whh_ref[...], preferred_element_type=jnp.float32) \
        + bhh_ref[...]
    i_r, i_z, i_n = gi[:, 0:H], gi[:, H:2 * H], gi[:, 2 * H:3 * H]
    h_r, h_z, h_n = gh[:, 0:H], gh[:, H:2 * H], gh[:, 2 * H:3 * H]
    r = jax.nn.sigmoid(i_r + h_r)
    z = jax.nn.sigmoid(i_z + h_z)
    n = jnp.tanh(i_n + r * h_n)
    h = (1.0 - z) * n + z * h
  pred_ref[...] = h


_ROWS = T * NP // BN

_tc_first = pl.pallas_call(
    _tc_first_body,
    grid=(_ROWS,),
    in_specs=[
        pl.BlockSpec((BN, D), lambda i: (i, 0)),
        pl.BlockSpec((BN, H), lambda i: (i, 0)),
        pl.BlockSpec((D, H), lambda i: (0, 0)),
    ],
    out_specs=pl.BlockSpec((BN, H), lambda i: (i, 0)),
    out_shape=jax.ShapeDtypeStruct((T * NP, H), jnp.float32),
    compiler_params=pltpu.CompilerParams(
        dimension_semantics=("parallel",)),
)

_tc_layer = pl.pallas_call(
    _tc_layer_body,
    grid=(_ROWS,),
    in_specs=[
        pl.BlockSpec((BN, H), lambda i: (i, 0)),
        pl.BlockSpec((BN, H), lambda i: (i, 0)),
        pl.BlockSpec((H, H), lambda i: (0, 0)),
        pl.BlockSpec((1, H), lambda i: (0, 0)),
    ],
    out_specs=pl.BlockSpec((BN, H), lambda i: (i, 0)),
    out_shape=jax.ShapeDtypeStruct((T * NP, H), jnp.float32),
    compiler_params=pltpu.CompilerParams(
        dimension_semantics=("parallel",)),
)

_BR = 80


def _tc_red_body(s16_ref, out_ref):
  out_ref[...] = jnp.sum(s16_ref[...], axis=-1)


_tc_red = pl.pallas_call(
    _tc_red_body,
    grid=(PP // 128 // _BR,),
    in_specs=[pl.BlockSpec((_BR, 128, 16), lambda i: (i, 0, 0))],
    out_specs=pl.BlockSpec((_BR, 128), lambda i: (i, 0)),
    out_shape=jax.ShapeDtypeStruct((PP // 128, 128), jnp.float32),
    compiler_params=pltpu.CompilerParams(
        dimension_semantics=("parallel",)),
)

_tc_gru = pl.pallas_call(
    _tc_gru_body,
    grid=(NP // BN,),
    in_specs=[
        pl.BlockSpec((T, BN, H), lambda i: (0, i, 0)),
        pl.BlockSpec((T, BN, H), lambda i: (0, i, 0)),
        pl.BlockSpec((1, H), lambda i: (0, 0)),
        pl.BlockSpec((H, 3 * H), lambda i: (0, 0)),
        pl.BlockSpec((H, 3 * H), lambda i: (0, 0)),
        pl.BlockSpec((1, 3 * H), lambda i: (0, 0)),
        pl.BlockSpec((1, 3 * H), lambda i: (0, 0)),
    ],
    out_specs=pl.BlockSpec((BN, H), lambda i: (i, 0)),
    out_shape=jax.ShapeDtypeStruct((NP, H), jnp.float32),
    compiler_params=pltpu.CompilerParams(
        dimension_semantics=("parallel",)),
)


@jax.jit
def _run(x_seq, edge_index, edge_pairs, W1, b1, W2, b2, W3, b3, W_ih, W_hh,
         b_ih, b_hh):
  x = jnp.pad(x_seq.astype(jnp.float32), ((0, 0), (0, NP - N), (0, 0)))
  ei = edge_index.astype(jnp.int32)
  src = jnp.pad(ei[:, 0, :], ((0, 0), (0, EP - E)), constant_values=N)
  dst = jnp.pad(ei[:, 1, :], ((0, 0), (0, EP - E)), constant_values=N)
  pairs = edge_pairs.astype(jnp.int32)
  psrc = jnp.pad(pairs[0], (0, PP - P))
  pdst = jnp.pad(pairs[1], (0, PP - P))
  ones = jnp.ones((NP, H), jnp.float32)
  srcS = src.reshape(T, NS, S_CHUNKS, SC_)
  dstS = dst.reshape(T, NS, S_CHUNKS, SC_)
  dstD = dst.reshape(T, NS, E_CHUNKS, EC)

  deg = _deg_kernel(dstD, ones)
  deg_flat = deg.reshape(T * NP, H)

  u = _tc_first(x.reshape(T * NP, D), deg_flat, W1)
  agg = _spmm_kernel(u.reshape(T, NP, H), srcS, dstS)

  u = _tc_layer(agg.reshape(T * NP, H), deg_flat, W2, b1.reshape(1, H))
  agg = _spmm_kernel(u.reshape(T, NP, H), srcS, dstS)

  u = _tc_layer(agg.reshape(T * NP, H), deg_flat, W3, b2.reshape(1, H))
  agg = _spmm_kernel(u.reshape(T, NP, H), srcS, dstS)

  pred = _tc_gru(agg, deg, b3.reshape(1, H),
                 W_ih.T, W_hh.T,
                 b_ih.reshape(1, 3 * H), b_hh.reshape(1, 3 * H))

  s16 = _score_kernel(pred, psrc, pdst)
  scores = _tc_red(s16.reshape(PP // 128, 128, 16))
  return scores.reshape(PP)[:P]


def kernel(x_seq, edge_index, edge_pairs, W1, b1, W2, b2, W3, b3, W_ih, W_hh,
           b_ih, b_hh):
  return _run(x_seq, edge_index, edge_pairs, W1, b1, W2, b2, W3, b3,
              W_ih, W_hh, b_ih, b_hh)


# speedup vs baseline: 1.0298x; 1.0168x over previous
import functools

import jax
import jax.numpy as jnp
from jax import lax
from jax.experimental import pallas as pl
from jax.experimental.pallas import tpu as pltpu
from jax.experimental.pallas import tpu_sc as plsc

N = 10000
D = 128
H = 128
T = 4
E = 320000
P = 100000

NC = 2
NS = 16
LANES = 16

NP = 10240
EP = 327680
PP = 102400

EC = 128
E_TILE = EP // NS
E_CHUNKS = E_TILE // EC
ROWS_TILE = NP // NS
P_TILE = PP // (NC * NS)
P_CHUNKS = P_TILE // EC

SC_ = 80
S_CHUNKS = E_TILE // SC_
S_GRP = 32
S_RING = 4

BN = 1024


_DEG_GRP = 8
_GRP = 32


def _deg_body(dst_hbm, ones_hbm, deg_hbm, didx_all, ones_v, acc_sh, sem):
  c = lax.axis_index("c")
  s = lax.axis_index("s")
  pltpu.sync_copy(ones_hbm.at[pl.ds(0, EC)], ones_v)

  for tt in range(T // NC):
    t = c * (T // NC) + tt
    pltpu.sync_copy(dst_hbm.at[t, s], didx_all)
    pltpu.sync_copy(ones_hbm.at[pl.ds(s * ROWS_TILE, ROWS_TILE)],
                    acc_sh.at[pl.ds(s * ROWS_TILE, ROWS_TILE)])
    plsc.subcore_barrier()

    def group(g, carry):
      for k in range(_DEG_GRP):
        pltpu.async_copy(ones_v, acc_sh.at[didx_all.at[g * _DEG_GRP + k]],
                         sem, add=True)
      for k in range(_DEG_GRP):
        pltpu.make_async_copy(ones_v, acc_sh.at[didx_all.at[0]],
                              sem).wait()
      return carry

    lax.fori_loop(0, E_CHUNKS // _DEG_GRP, group, 0)
    plsc.subcore_barrier()
    pltpu.sync_copy(acc_sh.at[pl.ds(s * ROWS_TILE, ROWS_TILE)],
                    deg_hbm.at[t, pl.ds(s * ROWS_TILE, ROWS_TILE)])
    plsc.subcore_barrier()


def _spmm_body(u_hbm, src_hbm, dst_hbm, agg_hbm, sidx_g, didx_g,
               r0, r1, r2, r3, acc_sh,
               g0, g1, g2, g3, s0, s1, s2, s3):
  c = lax.axis_index("c")
  s = lax.axis_index("s")
  rows = [r0, r1, r2, r3]
  gsem = [g0, g1, g2, g3]
  ssem = [s0, s1, s2, s3]

  for tt in range(T // NC):
    t = c * (T // NC) + tt
    pltpu.sync_copy(u_hbm.at[t, pl.ds(s * ROWS_TILE, ROWS_TILE)],
                    acc_sh.at[pl.ds(s * ROWS_TILE, ROWS_TILE)])
    plsc.subcore_barrier()

    ut = u_hbm.at[t]

    def fire_gather(lj, k):
      pltpu.async_copy(ut.at[sidx_g.at[lj]], rows[k], gsem[k])

    def wait_gather(k):
      pltpu.make_async_copy(ut.at[sidx_g.at[0]], rows[k], gsem[k]).wait()

    def fire_scatter(lj, k):
      pltpu.async_copy(rows[k], acc_sh.at[didx_g.at[lj]], ssem[k], add=True)

    def wait_scatter(k):
      pltpu.make_async_copy(rows[k], acc_sh.at[didx_g.at[0]],
                            ssem[k]).wait()

    def group(gg, carry):
      pltpu.sync_copy(src_hbm.at[t, s, pl.ds(gg * S_GRP, S_GRP)], sidx_g)
      pltpu.sync_copy(dst_hbm.at[t, s, pl.ds(gg * S_GRP, S_GRP)], didx_g)
      for k in range(S_RING):
        fire_gather(k, k)

      def inner(ii, carry2):
        base = S_RING * ii
        for k in range(S_RING):
          wait_gather(k)
          fire_scatter(base + k, k)
        for k in range(S_RING):
          wait_scatter(k)
          fire_gather(base + S_RING + k, k)
        return carry2

      lax.fori_loop(0, S_GRP // S_RING - 1, inner, 0)
      for k in range(S_RING):
        wait_gather(k)
        fire_scatter(S_GRP - S_RING + k, k)
      for k in range(S_RING):
        wait_scatter(k)
      return carry

    lax.fori_loop(0, S_CHUNKS // S_GRP, group, 0)
    plsc.subcore_barrier()
    pltpu.sync_copy(acc_sh.at[pl.ds(s * ROWS_TILE, ROWS_TILE)],
                    agg_hbm.at[t, pl.ds(s * ROWS_TILE, ROWS_TILE)])
    plsc.subcore_barrier()


def _score_body(pred_hbm, ps_hbm, pd_hbm, out_hbm,
                aidx0, bidx0, aidx1, bidx1,
                ar0, br0, ar1, br1, ov0, ov1,
                gs0, gs1, ws0, ws1):
  c = lax.axis_index("c")
  s = lax.axis_index("s")
  w = s * NC + c
  tbase = w * P_TILE
  aidx = [aidx0, aidx1]
  bidx = [bidx0, bidx1]
  ar = [ar0, ar1]
  br = [br0, br1]
  ov = [ov0, ov1]
  gs = [gs0, gs1]
  ws = [ws0, ws1]

  def fire(j, k):
    pltpu.sync_copy(ps_hbm.at[pl.ds(tbase + j * EC, EC)], aidx[k])
    pltpu.sync_copy(pd_hbm.at[pl.ds(tbase + j * EC, EC)], bidx[k])
    pltpu.async_copy(pred_hbm.at[aidx[k]], ar[k], gs[k])
    pltpu.async_copy(pred_hbm.at[bidx[k]], br[k], gs[k])

  def compute(j, k):
    pltpu.make_async_copy(pred_hbm.at[aidx[k]], ar[k], gs[k]).wait()
    pltpu.make_async_copy(pred_hbm.at[bidx[k]], br[k], gs[k]).wait()

    def pgroup(g, carry2):
      p0 = g * 8
      for kk in range(8):
        p = p0 + kk
        acc = ar[k][p, pl.ds(0, LANES)] * br[k][p, pl.ds(0, LANES)]
        for q in range(1, H // LANES):
          acc = acc + ar[k][p, pl.ds(q * LANES, LANES)] * \
              br[k][p, pl.ds(q * LANES, LANES)]
        ov[k][p, :] = acc
      return carry2

    lax.fori_loop(0, EC // 8, pgroup, 0)
    pltpu.async_copy(ov[k], out_hbm.at[pl.ds(tbase + j * EC, EC)], ws[k])

  fire(0, 0)

  def body(jj, carry):
    a = 2 * jj
    b = a + 1
    fire(b, 1)
    compute(a, 0)
    pltpu.make_async_copy(ov[0], out_hbm.at[pl.ds(tbase, EC)], ws[0]).wait()
    fire(a + 2, 0)
    compute(b, 1)
    pltpu.make_async_copy(ov[1], out_hbm.at[pl.ds(tbase, EC)], ws[1]).wait()
    return carry

  lax.fori_loop(0, (P_CHUNKS - 1) // 2, body, 0)
  compute(P_CHUNKS - 1, 0)
  pltpu.make_async_copy(ov[0], out_hbm.at[pl.ds(tbase, EC)], ws[0]).wait()


_sc_mesh = plsc.VectorSubcoreMesh(core_axis_name="c", subcore_axis_name="s")

_deg_kernel = pl.kernel(
    _deg_body,
    out_type=jax.ShapeDtypeStruct((T, NP, H), jnp.float32),
    mesh=_sc_mesh,
    scratch_types=[
        pltpu.VMEM((E_CHUNKS, EC), jnp.int32),
        pltpu.VMEM((EC, H), jnp.float32),
        pltpu.VMEM_SHARED((NP, H), jnp.float32),
        pltpu.SemaphoreType.DMA,
    ],
)

_spmm_kernel = pl.kernel(
    _spmm_body,
    out_type=jax.ShapeDtypeStruct((T, NP, H), jnp.float32),
    mesh=_sc_mesh,
    scratch_types=[
        pltpu.VMEM((S_GRP, SC_), jnp.int32),
        pltpu.VMEM((S_GRP, SC_), jnp.int32),
    ] + [pltpu.VMEM((SC_, H), jnp.float32)] * 4
    + [pltpu.VMEM_SHARED((NP, H), jnp.float32)]
    + [pltpu.SemaphoreType.DMA] * 8,
)

_score_kernel = pl.kernel(
    _score_body,
    out_type=jax.ShapeDtypeStruct((PP, 16), jnp.float32),
    mesh=_sc_mesh,
    scratch_types=[
        pltpu.VMEM((EC,), jnp.int32),
        pltpu.VMEM((EC,), jnp.int32),
        pltpu.VMEM((EC,), jnp.int32),
        pltpu.VMEM((EC,), jnp.int32),
        pltpu.VMEM((EC, H), jnp.float32),
        pltpu.VMEM((EC, H), jnp.float32),
        pltpu.VMEM((EC, H), jnp.float32),
        pltpu.VMEM((EC, H), jnp.float32),
        pltpu.VMEM((EC, 16), jnp.float32),
        pltpu.VMEM((EC, 16), jnp.float32),
        pltpu.SemaphoreType.DMA,
        pltpu.SemaphoreType.DMA,
        pltpu.SemaphoreType.DMA,
        pltpu.SemaphoreType.DMA,
    ],
)


def _dis_from_deg(deg_blk):
  deg = deg_blk[:, 0:1]
  return lax.rsqrt(jnp.maximum(deg, 1e-12)).astype(jnp.float32)


def _tc_first_body(x_ref, deg_ref, w_ref, u_ref):
  dis = _dis_from_deg(deg_ref[...])
  h = jnp.dot(x_ref[...], w_ref[...], preferred_element_type=jnp.float32)
  u_ref[...] = dis * h


def _tc_layer_body(agg_ref, deg_ref, w_ref, b_ref, u_ref):
  dis = _dis_from_deg(deg_ref[...])
  h = jax.nn.relu(dis * agg_ref[...] + b_ref[...])
  u_ref[...] = dis * jnp.dot(h, w_ref[...], preferred_element_type=jnp.float32)


def _tc_gru_body(agg_ref, deg_ref, b3_ref, wih_ref, whh_ref, bih_ref, bhh_ref,
                 pred_ref):
  h = jnp.zeros((BN, H), jnp.float32)
  for t in range(T):
    dis = _dis_from_deg(deg_ref[t])
    xt = dis * agg_ref[t] + b3_ref[...]
    gi = jnp.dot(xt, wih_ref[...], preferred_element_type=jnp.float32) \
        + bih_ref[...]
    gh = jnp.dot(h, whh_ref[...], preferred_element_type=jnp.float32) \
        + bhh_ref[...]
    i_r, i_z, i_n = gi[:, 0:H], gi[:, H:2 * H], gi[:, 2 * H:3 * H]
    h_r, h_z, h_n = gh[:, 0:H], gh[:, H:2 * H], gh[:, 2 * H:3 * H]
    r = jax.nn.sigmoid(i_r + h_r)
    z = jax.nn.sigmoid(i_z + h_z)
    n = jnp.tanh(i_n + r * h_n)
    h = (1.0 - z) * n + z * h
  pred_ref[...] = h


_ROWS = T * NP // BN

_tc_first = pl.pallas_call(
    _tc_first_body,
    grid=(_ROWS,),
    in_specs=[
        pl.BlockSpec((BN, D), lambda i: (i, 0)),
        pl.BlockSpec((BN, H), lambda i: (i, 0)),
        pl.BlockSpec((D, H), lambda i: (0, 0)),
    ],
    out_specs=pl.BlockSpec((BN, H), lambda i: (i, 0)),
    out_shape=jax.ShapeDtypeStruct((T * NP, H), jnp.float32),
    compiler_params=pltpu.CompilerParams(
        dimension_semantics=("parallel",)),
)

_tc_layer = pl.pallas_call(
    _tc_layer_body,
    grid=(_ROWS,),
    in_specs=[
        pl.BlockSpec((BN, H), lambda i: (i, 0)),
        pl.BlockSpec((BN, H), lambda i: (i, 0)),
        pl.BlockSpec((H, H), lambda i: (0, 0)),
        pl.BlockSpec((1, H), lambda i: (0, 0)),
    ],
    out_specs=pl.BlockSpec((BN, H), lambda i: (i, 0)),
    out_shape=jax.ShapeDtypeStruct((T * NP, H), jnp.float32),
    compiler_params=pltpu.CompilerParams(
        dimension_semantics=("parallel",)),
)

_BR = 80


def _tc_red_body(s16_ref, out_ref):
  out_ref[...] = jnp.sum(s16_ref[...], axis=-1)


_tc_red = pl.pallas_call(
    _tc_red_body,
    grid=(PP // 128 // _BR,),
    in_specs=[pl.BlockSpec((_BR, 128, 16), lambda i: (i, 0, 0))],
    out_specs=pl.BlockSpec((_BR, 128), lambda i: (i, 0)),
    out_shape=jax.ShapeDtypeStruct((PP // 128, 128), jnp.float32),
    compiler_params=pltpu.CompilerParams(
        dimension_semantics=("parallel",)),
)

_tc_gru = pl.pallas_call(
    _tc_gru_body,
    grid=(NP // BN,),
    in_specs=[
        pl.BlockSpec((T, BN, H), lambda i: (0, i, 0)),
        pl.BlockSpec((T, BN, H), lambda i: (0, i, 0)),
        pl.BlockSpec((1, H), lambda i: (0, 0)),
        pl.BlockSpec((H, 3 * H), lambda i: (0, 0)),
        pl.BlockSpec((H, 3 * H), lambda i: (0, 0)),
        pl.BlockSpec((1, 3 * H), lambda i: (0, 0)),
        pl.BlockSpec((1, 3 * H), lambda i: (0, 0)),
    ],
    out_specs=pl.BlockSpec((BN, H), lambda i: (i, 0)),
    out_shape=jax.ShapeDtypeStruct((NP, H), jnp.float32),
    compiler_params=pltpu.CompilerParams(
        dimension_semantics=("parallel",)),
)


@jax.jit
def _run(x_seq, edge_index, edge_pairs, W1, b1, W2, b2, W3, b3, W_ih, W_hh,
         b_ih, b_hh):
  x = jnp.pad(x_seq.astype(jnp.float32), ((0, 0), (0, NP - N), (0, 0)))
  ei = edge_index.astype(jnp.int32)
  src = jnp.pad(ei[:, 0, :], ((0, 0), (0, EP - E)), constant_values=N)
  dst = jnp.pad(ei[:, 1, :], ((0, 0), (0, EP - E)), constant_values=N)
  pairs = edge_pairs.astype(jnp.int32)
  psrc = jnp.pad(pairs[0], (0, PP - P))
  pdst = jnp.pad(pairs[1], (0, PP - P))
  ones = jnp.ones((NP, H), jnp.float32)
  srcS = src.reshape(T, NS, S_CHUNKS, SC_)
  dstS = dst.reshape(T, NS, S_CHUNKS, SC_)
  dstD = dst.reshape(T, NS, E_CHUNKS, EC)

  deg = _deg_kernel(dstD, ones)
  deg_flat = deg.reshape(T * NP, H)

  u = _tc_first(x.reshape(T * NP, D), deg_flat, W1)
  agg = _spmm_kernel(u.reshape(T, NP, H), srcS, dstS)

  u = _tc_layer(agg.reshape(T * NP, H), deg_flat, W2, b1.reshape(1, H))
  agg = _spmm_kernel(u.reshape(T, NP, H), srcS, dstS)

  u = _tc_layer(agg.reshape(T * NP, H), deg_flat, W3, b2.reshape(1, H))
  agg = _spmm_kernel(u.reshape(T, NP, H), srcS, dstS)

  pred = _tc_gru(agg, deg, b3.reshape(1, H),
                 W_ih.T, W_hh.T,
                 b_ih.reshape(1, 3 * H), b_hh.reshape(1, 3 * H))

  s16 = _score_kernel(pred, psrc, pdst)
  scores = _tc_red(s16.reshape(PP // 128, 128, 16))
  return scores.reshape(PP)[:P]


def kernel(x_seq, edge_index, edge_pairs, W1, b1, W2, b2, W3, b3, W_ih, W_hh,
           b_ih, b_hh):
  return _run(x_seq, edge_index, edge_pairs, W1, b1, W2, b2, W3, b3,
              W_ih, W_hh, b_ih, b_hh)
